# Initial kernel scaffold; baseline (speedup 1.0000x reference)
import jax
import jax.numpy as jnp
from jax.experimental import pallas as pl


def kernel(ctr_feats, ctr_preds, ctr_origins, sa_pts_list, gt_boxes_list, gt_labels_list, W1b, b1b, g1b, beta1b, W2b, b2b, W1c, b1c, g1c, beta1c, W2c, b2c, mean_size):
    raise NotImplementedError("write your pallas kernel here")


# speedup vs baseline: 1.9166x; 1.9166x over previous
import numpy as np
import jax
import jax.numpy as jnp
from jax.experimental import pallas as pl

BIN_SIZE = 12
BIN_INTER = 2.0 * np.pi / BIN_SIZE
CODE_SIZE = 6 + 2 * BIN_SIZE
MID = 128

N_BLK = 4096


def _fused_kernel(x_ref, pts_ref, w1_ref, b1_ref, w2b_ref, b2b_ref,
                  w2c_ref, b2c_ref, ms_ref, box_ref, cls_ref, dec_ref):
    x = x_ref[0]
    w1 = w1_ref[...]
    h = jax.lax.dot_general(w1, x, (((1,), (0,)), ((), ())),
                            preferred_element_type=jnp.float32)
    h = jnp.maximum(h + b1_ref[...], 0.0)
    hb = h[:MID]
    hc = h[MID:]

    box = jax.lax.dot_general(hb, w2b_ref[...], (((0,), (1,)), ((), ())),
                              preferred_element_type=jnp.float32)
    box = box + b2b_ref[...]
    cls = jax.lax.dot_general(hc, w2c_ref[...], (((0,), (1,)), ((), ())),
                              preferred_element_type=jnp.float32)
    cls = cls + b2c_ref[...]
    box_ref[0] = box
    cls_ref[0] = cls

    c0, c1, c2 = cls[:, 0:1], cls[:, 1:2], cls[:, 2:3]
    pc = jnp.where(c0 >= jnp.maximum(c1, c2), 0,
                   jnp.where(c1 >= c2, 1, 2))
    iota3 = jax.lax.broadcasted_iota(jnp.int32, (1, 3), 1)
    onehot = (pc == iota3).astype(jnp.float32)
    anchor = jax.lax.dot_general(onehot, ms_ref[...], (((1,), (0,)), ((), ())),
                                 preferred_element_type=jnp.float32)
    dxa, dya, dza = anchor[:, 0:1], anchor[:, 1:2], anchor[:, 2:3]

    pts = pts_ref[0]
    xa, ya, za = pts[:, 0:1], pts[:, 1:2], pts[:, 2:3]
    xt, yt, zt = box[:, 0:1], box[:, 1:2], box[:, 2:3]
    dxt, dyt, dzt = box[:, 3:4], box[:, 4:5], box[:, 5:6]

    diagonal = jnp.sqrt(dxa * dxa + dya * dya)
    xg = xt * diagonal + xa
    yg = yt * diagonal + ya
    zg = zt * dza + za
    dxg = jnp.exp(dxt) * dxa
    dyg = jnp.exp(dyt) * dya
    dzg = jnp.exp(dzt) * dza

    bb = box[:, 6:6 + BIN_SIZE]
    rr = box[:, 6 + BIN_SIZE:]
    best = jnp.max(bb, axis=1, keepdims=True)
    iota12 = jax.lax.broadcasted_iota(jnp.int32, (1, BIN_SIZE), 1)
    bin_id = jnp.min(jnp.where(bb >= best, iota12, BIN_SIZE),
                     axis=1, keepdims=True)
    bin_res = jnp.sum(jnp.where(iota12 == bin_id, rr, 0.0),
                      axis=1, keepdims=True)
    rg = bin_id.astype(jnp.float32) * BIN_INTER + bin_res

    dec_ref[0] = jnp.concatenate([xg, yg, zg, dxg, dyg, dzg, rg], axis=1)


def kernel(ctr_feats, ctr_preds, ctr_origins, sa_pts_list, gt_boxes_list,
           gt_labels_list, W1b, b1b, g1b, beta1b, W2b, b2b, W1c, b1c, g1c,
           beta1c, W2c, b2c, mean_size):
    B, C, N = ctr_feats.shape

    w1 = jnp.concatenate([W1b * g1b[:, None], W1c * g1c[:, None]], axis=0)
    b1 = jnp.concatenate([b1b * g1b + beta1b, b1c * g1c + beta1c])[:, None]

    grid = (B, N // N_BLK)
    out_shapes = (
        jax.ShapeDtypeStruct((B, N, CODE_SIZE), jnp.float32),
        jax.ShapeDtypeStruct((B, N, 3), jnp.float32),
        jax.ShapeDtypeStruct((B, N, 7), jnp.float32),
    )
    full = lambda shape: pl.BlockSpec(shape, lambda b, n: (0,) * len(shape))
    ctr_box_preds, pt_cls_preds, pt_box_preds = pl.pallas_call(
        _fused_kernel,
        grid=grid,
        in_specs=[
            pl.BlockSpec((1, C, N_BLK), lambda b, n: (b, 0, n)),
            pl.BlockSpec((1, N_BLK, 3), lambda b, n: (b, n, 0)),
            full((2 * MID, C)),
            full((2 * MID, 1)),
            full((CODE_SIZE, MID)),
            full((1, CODE_SIZE)),
            full((3, MID)),
            full((1, 3)),
            full((3, 3)),
        ],
        out_specs=[
            pl.BlockSpec((1, N_BLK, CODE_SIZE), lambda b, n: (b, n, 0)),
            pl.BlockSpec((1, N_BLK, 3), lambda b, n: (b, n, 0)),
            pl.BlockSpec((1, N_BLK, 7), lambda b, n: (b, n, 0)),
        ],
        out_shape=out_shapes,
    )(ctr_feats, ctr_preds, w1, b1, W2b, b2b[None, :], W2c, b2c[None, :],
      mean_size)
    return (ctr_box_preds, pt_cls_preds, pt_box_preds)


# speedup vs baseline: 3.5325x; 1.8431x over previous
import numpy as np
import jax
import jax.numpy as jnp
from jax.experimental import pallas as pl
from jax.experimental.pallas import tpu as pltpu

BIN_SIZE = 12
BIN_INTER = 2.0 * np.pi / BIN_SIZE
CODE_SIZE = 6 + 2 * BIN_SIZE
MID = 128

N_BLK = 4096


def _fused_kernel(ms_ref, x_ref, pts_ref, w1_ref, b1_ref, w2b_ref, b2b_ref,
                  w2c_ref, b2c_ref, box_ref, cls_ref, dec_ref):
    x = x_ref[0]
    w1 = w1_ref[...]
    h = jax.lax.dot_general(w1, x, (((1,), (0,)), ((), ())),
                            preferred_element_type=jnp.float32)
    h = jnp.maximum(h + b1_ref[...], 0.0)
    hb = h[:MID]
    hc = h[MID:]

    box = jax.lax.dot_general(w2b_ref[...], hb, (((1,), (0,)), ((), ())),
                              preferred_element_type=jnp.float32)
    box = box + b2b_ref[...]
    cls = jax.lax.dot_general(w2c_ref[...], hc, (((1,), (0,)), ((), ())),
                              preferred_element_type=jnp.float32)
    cls = cls + b2c_ref[...]
    box_ref[0] = box.T
    cls_ref[0] = cls.T

    c0, c1, c2 = cls[0:1], cls[1:2], cls[2:3]
    is0 = c0 >= jnp.maximum(c1, c2)
    is1 = c1 >= c2
    dxa = jnp.where(is0, ms_ref[0, 0], jnp.where(is1, ms_ref[1, 0], ms_ref[2, 0]))
    dya = jnp.where(is0, ms_ref[0, 1], jnp.where(is1, ms_ref[1, 1], ms_ref[2, 1]))
    dza = jnp.where(is0, ms_ref[0, 2], jnp.where(is1, ms_ref[1, 2], ms_ref[2, 2]))

    pts = pts_ref[0]
    xa, ya, za = pts[0:1], pts[1:2], pts[2:3]
    xt, yt, zt, dxt, dyt, dzt = (box[i:i + 1] for i in range(6))

    diagonal = jnp.sqrt(dxa * dxa + dya * dya)
    xg = xt * diagonal + xa
    yg = yt * diagonal + ya
    zg = zt * dza + za
    dxg = jnp.exp(dxt) * dxa
    dyg = jnp.exp(dyt) * dya
    dzg = jnp.exp(dzt) * dza

    bb = box[6:6 + BIN_SIZE]
    rr = box[6 + BIN_SIZE:]
    best = jnp.max(bb, axis=0, keepdims=True)
    iota12 = jax.lax.broadcasted_iota(jnp.int32, (BIN_SIZE, 1), 0)
    bin_id = jnp.min(jnp.where(bb >= best, iota12, BIN_SIZE),
                     axis=0, keepdims=True)
    bin_res = jnp.sum(jnp.where(iota12 == bin_id, rr, 0.0),
                      axis=0, keepdims=True)
    rg = bin_id.astype(jnp.float32) * BIN_INTER + bin_res

    dec = jnp.concatenate([xg, yg, zg, dxg, dyg, dzg, rg], axis=0)
    dec_ref[0] = dec.T


def kernel(ctr_feats, ctr_preds, ctr_origins, sa_pts_list, gt_boxes_list,
           gt_labels_list, W1b, b1b, g1b, beta1b, W2b, b2b, W1c, b1c, g1c,
           beta1c, W2c, b2c, mean_size):
    B, C, N = ctr_feats.shape

    w1 = jnp.concatenate([W1b * g1b[:, None], W1c * g1c[:, None]], axis=0)
    b1 = jnp.concatenate([b1b * g1b + beta1b, b1c * g1c + beta1c])[:, None]
    pts_t = jnp.transpose(ctr_preds, (0, 2, 1))

    grid = (B, N // N_BLK)
    out_shapes = (
        jax.ShapeDtypeStruct((B, N, CODE_SIZE), jnp.float32),
        jax.ShapeDtypeStruct((B, N, 3), jnp.float32),
        jax.ShapeDtypeStruct((B, N, 7), jnp.float32),
    )
    full = lambda shape: pl.BlockSpec(shape, lambda b, n: (0,) * len(shape))
    ctr_box_preds, pt_cls_preds, pt_box_preds = pl.pallas_call(
        _fused_kernel,
        grid=grid,
        in_specs=[
            pl.BlockSpec(memory_space=pltpu.SMEM),
            pl.BlockSpec((1, C, N_BLK), lambda b, n: (b, 0, n)),
            pl.BlockSpec((1, 3, N_BLK), lambda b, n: (b, 0, n)),
            full((2 * MID, C)),
            full((2 * MID, 1)),
            full((CODE_SIZE, MID)),
            full((CODE_SIZE, 1)),
            full((3, MID)),
            full((3, 1)),
        ],
        out_specs=[
            pl.BlockSpec((1, N_BLK, CODE_SIZE), lambda b, n: (b, n, 0)),
            pl.BlockSpec((1, N_BLK, 3), lambda b, n: (b, n, 0)),
            pl.BlockSpec((1, N_BLK, 7), lambda b, n: (b, n, 0)),
        ],
        out_shape=out_shapes,
    )(mean_size, ctr_feats, pts_t, w1, b1, W2b, b2b[:, None], W2c, b2c[:, None])
    return (ctr_box_preds, pt_cls_preds, pt_box_preds)


# speedup vs baseline: 3.6341x; 1.0288x over previous
import numpy as np
import jax
import jax.numpy as jnp
from jax.experimental import pallas as pl
from jax.experimental.pallas import tpu as pltpu

BIN_SIZE = 12
BIN_INTER = 2.0 * np.pi / BIN_SIZE
CODE_SIZE = 6 + 2 * BIN_SIZE
MID = 128

N_BLK = 8192


def _fused_kernel(ms_ref, x_ref, pts_ref, w1_ref, b1_ref, w2b_ref, b2b_ref,
                  w2c_ref, b2c_ref, box_ref, cls_ref, dec_ref):
    x = x_ref[0]
    w1 = w1_ref[...]
    h = jax.lax.dot_general(w1, x, (((1,), (0,)), ((), ())),
                            preferred_element_type=jnp.float32)
    h = jnp.maximum(h + b1_ref[...], 0.0)
    hb = h[:MID]
    hc = h[MID:]

    box = jax.lax.dot_general(w2b_ref[...], hb, (((1,), (0,)), ((), ())),
                              preferred_element_type=jnp.float32)
    box = box + b2b_ref[...]
    cls = jax.lax.dot_general(w2c_ref[...], hc, (((1,), (0,)), ((), ())),
                              preferred_element_type=jnp.float32)
    cls = cls + b2c_ref[...]
    box_ref[0] = box.T
    cls_ref[0] = cls.T

    c0, c1, c2 = cls[0:1], cls[1:2], cls[2:3]
    is0 = c0 >= jnp.maximum(c1, c2)
    is1 = c1 >= c2
    dxa = jnp.where(is0, ms_ref[0, 0], jnp.where(is1, ms_ref[1, 0], ms_ref[2, 0]))
    dya = jnp.where(is0, ms_ref[0, 1], jnp.where(is1, ms_ref[1, 1], ms_ref[2, 1]))
    dza = jnp.where(is0, ms_ref[0, 2], jnp.where(is1, ms_ref[1, 2], ms_ref[2, 2]))

    pts = pts_ref[0]
    xa, ya, za = pts[0:1], pts[1:2], pts[2:3]
    xt, yt, zt, dxt, dyt, dzt = (box[i:i + 1] for i in range(6))

    diagonal = jnp.sqrt(dxa * dxa + dya * dya)
    xg = xt * diagonal + xa
    yg = yt * diagonal + ya
    zg = zt * dza + za
    dxg = jnp.exp(dxt) * dxa
    dyg = jnp.exp(dyt) * dya
    dzg = jnp.exp(dzt) * dza

    bb = box[6:6 + BIN_SIZE]
    rr = box[6 + BIN_SIZE:]
    best = jnp.max(bb, axis=0, keepdims=True)
    iota12 = jax.lax.broadcasted_iota(jnp.int32, (BIN_SIZE, 1), 0)
    bin_id = jnp.min(jnp.where(bb >= best, iota12, BIN_SIZE),
                     axis=0, keepdims=True)
    bin_res = jnp.sum(jnp.where(iota12 == bin_id, rr, 0.0),
                      axis=0, keepdims=True)
    rg = bin_id.astype(jnp.float32) * BIN_INTER + bin_res

    dec = jnp.concatenate([xg, yg, zg, dxg, dyg, dzg, rg], axis=0)
    dec_ref[0] = dec.T


def kernel(ctr_feats, ctr_preds, ctr_origins, sa_pts_list, gt_boxes_list,
           gt_labels_list, W1b, b1b, g1b, beta1b, W2b, b2b, W1c, b1c, g1c,
           beta1c, W2c, b2c, mean_size):
    B, C, N = ctr_feats.shape

    w1 = jnp.concatenate([W1b * g1b[:, None], W1c * g1c[:, None]], axis=0)
    b1 = jnp.concatenate([b1b * g1b + beta1b, b1c * g1c + beta1c])[:, None]
    pts_t = jnp.transpose(ctr_preds, (0, 2, 1))

    grid = (B, N // N_BLK)
    out_shapes = (
        jax.ShapeDtypeStruct((B, N, CODE_SIZE), jnp.float32),
        jax.ShapeDtypeStruct((B, N, 3), jnp.float32),
        jax.ShapeDtypeStruct((B, N, 7), jnp.float32),
    )
    full = lambda shape: pl.BlockSpec(shape, lambda b, n: (0,) * len(shape))
    ctr_box_preds, pt_cls_preds, pt_box_preds = pl.pallas_call(
        _fused_kernel,
        grid=grid,
        in_specs=[
            pl.BlockSpec(memory_space=pltpu.SMEM),
            pl.BlockSpec((1, C, N_BLK), lambda b, n: (b, 0, n)),
            pl.BlockSpec((1, 3, N_BLK), lambda b, n: (b, 0, n)),
            full((2 * MID, C)),
            full((2 * MID, 1)),
            full((CODE_SIZE, MID)),
            full((CODE_SIZE, 1)),
            full((3, MID)),
            full((3, 1)),
        ],
        out_specs=[
            pl.BlockSpec((1, N_BLK, CODE_SIZE), lambda b, n: (b, n, 0)),
            pl.BlockSpec((1, N_BLK, 3), lambda b, n: (b, n, 0)),
            pl.BlockSpec((1, N_BLK, 7), lambda b, n: (b, n, 0)),
        ],
        out_shape=out_shapes,
        compiler_params=pltpu.CompilerParams(
            dimension_semantics=("parallel", "parallel")),
    )(mean_size, ctr_feats, pts_t, w1, b1, W2b, b2b[:, None], W2c, b2c[:, None])
    return (ctr_box_preds, pt_cls_preds, pt_box_preds)


# speedup vs baseline: 9.4265x; 2.5939x over previous
import numpy as np
import jax
import jax.numpy as jnp
from jax.experimental import pallas as pl
from jax.experimental.pallas import tpu as pltpu

BIN_SIZE = 12
BIN_INTER = 2.0 * np.pi / BIN_SIZE
CODE_SIZE = 6 + 2 * BIN_SIZE
MID = 128

N_BLK = 8192


def _fused_kernel(ms_ref, x_ref, pts_ref, w1_ref, b1_ref, w2b_ref, b2b_ref,
                  w2c_ref, b2c_ref, box_ref, cls_ref, dec_ref):
    x = x_ref[0]
    w1 = w1_ref[...]
    h = jax.lax.dot_general(w1, x, (((1,), (0,)), ((), ())),
                            preferred_element_type=jnp.float32)
    h = jnp.maximum(h + b1_ref[...], 0.0)
    hb = h[:MID]
    hc = h[MID:]

    box = jax.lax.dot_general(w2b_ref[...], hb, (((1,), (0,)), ((), ())),
                              preferred_element_type=jnp.float32)
    box = box + b2b_ref[...]
    cls = jax.lax.dot_general(w2c_ref[...], hc, (((1,), (0,)), ((), ())),
                              preferred_element_type=jnp.float32)
    cls = cls + b2c_ref[...]
    box_ref[0] = box
    cls_ref[0] = cls

    c0, c1, c2 = cls[0:1], cls[1:2], cls[2:3]
    is0 = c0 >= jnp.maximum(c1, c2)
    is1 = c1 >= c2
    dxa = jnp.where(is0, ms_ref[0, 0], jnp.where(is1, ms_ref[1, 0], ms_ref[2, 0]))
    dya = jnp.where(is0, ms_ref[0, 1], jnp.where(is1, ms_ref[1, 1], ms_ref[2, 1]))
    dza = jnp.where(is0, ms_ref[0, 2], jnp.where(is1, ms_ref[1, 2], ms_ref[2, 2]))

    pts = pts_ref[0]
    xa, ya, za = pts[0:1], pts[1:2], pts[2:3]
    xt, yt, zt, dxt, dyt, dzt = (box[i:i + 1] for i in range(6))

    diagonal = jnp.sqrt(dxa * dxa + dya * dya)
    xg = xt * diagonal + xa
    yg = yt * diagonal + ya
    zg = zt * dza + za
    dxg = jnp.exp(dxt) * dxa
    dyg = jnp.exp(dyt) * dya
    dzg = jnp.exp(dzt) * dza

    bb = box[6:6 + BIN_SIZE]
    rr = box[6 + BIN_SIZE:]
    best = jnp.max(bb, axis=0, keepdims=True)
    iota12 = jax.lax.broadcasted_iota(jnp.int32, (BIN_SIZE, 1), 0)
    bin_id = jnp.min(jnp.where(bb >= best, iota12, BIN_SIZE),
                     axis=0, keepdims=True)
    bin_res = jnp.sum(jnp.where(iota12 == bin_id, rr, 0.0),
                      axis=0, keepdims=True)
    rg = bin_id.astype(jnp.float32) * BIN_INTER + bin_res

    dec = jnp.concatenate([xg, yg, zg, dxg, dyg, dzg, rg], axis=0)
    dec_ref[0] = dec


def kernel(ctr_feats, ctr_preds, ctr_origins, sa_pts_list, gt_boxes_list,
           gt_labels_list, W1b, b1b, g1b, beta1b, W2b, b2b, W1c, b1c, g1c,
           beta1c, W2c, b2c, mean_size):
    B, C, N = ctr_feats.shape

    w1 = jnp.concatenate([W1b * g1b[:, None], W1c * g1c[:, None]], axis=0)
    b1 = jnp.concatenate([b1b * g1b + beta1b, b1c * g1c + beta1c])[:, None]
    pts_t = jnp.transpose(ctr_preds, (0, 2, 1))

    grid = (B, N // N_BLK)
    out_shapes = (
        jax.ShapeDtypeStruct((B, CODE_SIZE, N), jnp.float32),
        jax.ShapeDtypeStruct((B, 3, N), jnp.float32),
        jax.ShapeDtypeStruct((B, 7, N), jnp.float32),
    )
    full = lambda shape: pl.BlockSpec(shape, lambda b, n: (0,) * len(shape))
    ctr_box_preds, pt_cls_preds, pt_box_preds = pl.pallas_call(
        _fused_kernel,
        grid=grid,
        in_specs=[
            pl.BlockSpec(memory_space=pltpu.SMEM),
            pl.BlockSpec((1, C, N_BLK), lambda b, n: (b, 0, n)),
            pl.BlockSpec((1, 3, N_BLK), lambda b, n: (b, 0, n)),
            full((2 * MID, C)),
            full((2 * MID, 1)),
            full((CODE_SIZE, MID)),
            full((CODE_SIZE, 1)),
            full((3, MID)),
            full((3, 1)),
        ],
        out_specs=[
            pl.BlockSpec((1, CODE_SIZE, N_BLK), lambda b, n: (b, 0, n)),
            pl.BlockSpec((1, 3, N_BLK), lambda b, n: (b, 0, n)),
            pl.BlockSpec((1, 7, N_BLK), lambda b, n: (b, 0, n)),
        ],
        out_shape=out_shapes,
        compiler_params=pltpu.CompilerParams(
            dimension_semantics=("parallel", "parallel")),
    )(mean_size, ctr_feats, pts_t, w1, b1, W2b, b2b[:, None], W2c, b2c[:, None])
    return (jnp.transpose(ctr_box_preds, (0, 2, 1)),
            jnp.transpose(pt_cls_preds, (0, 2, 1)),
            jnp.transpose(pt_box_preds, (0, 2, 1)))


# speedup vs baseline: 9.8054x; 1.0402x over previous
import numpy as np
import jax
import jax.numpy as jnp
from jax.experimental import pallas as pl
from jax.experimental.pallas import tpu as pltpu

BIN_SIZE = 12
BIN_INTER = 2.0 * np.pi / BIN_SIZE
CODE_SIZE = 6 + 2 * BIN_SIZE
MID = 128

N_BLK = 16384


def _fused_kernel(ms_ref, x_ref, pts_ref, w1_ref, b1_ref, w2b_ref, b2b_ref,
                  w2c_ref, b2c_ref, box_ref, cls_ref, dec_ref):
    x = x_ref[0]
    w1 = w1_ref[...]
    h = jax.lax.dot_general(w1, x, (((1,), (0,)), ((), ())),
                            preferred_element_type=jnp.float32)
    h = jnp.maximum(h + b1_ref[...], 0.0)
    hb = h[:MID]
    hc = h[MID:]

    box = jax.lax.dot_general(w2b_ref[...], hb, (((1,), (0,)), ((), ())),
                              preferred_element_type=jnp.float32)
    box = box + b2b_ref[...]
    cls = jax.lax.dot_general(w2c_ref[...], hc, (((1,), (0,)), ((), ())),
                              preferred_element_type=jnp.float32)
    cls = cls + b2c_ref[...]
    box_ref[0] = box
    cls_ref[0] = cls

    c0, c1, c2 = cls[0:1], cls[1:2], cls[2:3]
    is0 = c0 >= jnp.maximum(c1, c2)
    is1 = c1 >= c2
    dxa = jnp.where(is0, ms_ref[0, 0], jnp.where(is1, ms_ref[1, 0], ms_ref[2, 0]))
    dya = jnp.where(is0, ms_ref[0, 1], jnp.where(is1, ms_ref[1, 1], ms_ref[2, 1]))
    dza = jnp.where(is0, ms_ref[0, 2], jnp.where(is1, ms_ref[1, 2], ms_ref[2, 2]))

    pts = pts_ref[0]
    xa, ya, za = pts[0:1], pts[1:2], pts[2:3]
    xt, yt, zt, dxt, dyt, dzt = (box[i:i + 1] for i in range(6))

    diagonal = jnp.sqrt(dxa * dxa + dya * dya)
    xg = xt * diagonal + xa
    yg = yt * diagonal + ya
    zg = zt * dza + za
    dxg = jnp.exp(dxt) * dxa
    dyg = jnp.exp(dyt) * dya
    dzg = jnp.exp(dzt) * dza

    bb = box[6:6 + BIN_SIZE]
    rr = box[6 + BIN_SIZE:]
    best = jnp.max(bb, axis=0, keepdims=True)
    iota12 = jax.lax.broadcasted_iota(jnp.int32, (BIN_SIZE, 1), 0)
    bin_id = jnp.min(jnp.where(bb >= best, iota12, BIN_SIZE),
                     axis=0, keepdims=True)
    bin_res = jnp.sum(jnp.where(iota12 == bin_id, rr, 0.0),
                      axis=0, keepdims=True)
    rg = bin_id.astype(jnp.float32) * BIN_INTER + bin_res

    dec = jnp.concatenate([xg, yg, zg, dxg, dyg, dzg, rg], axis=0)
    dec_ref[0] = dec


def kernel(ctr_feats, ctr_preds, ctr_origins, sa_pts_list, gt_boxes_list,
           gt_labels_list, W1b, b1b, g1b, beta1b, W2b, b2b, W1c, b1c, g1c,
           beta1c, W2c, b2c, mean_size):
    B, C, N = ctr_feats.shape

    w1 = jnp.concatenate([W1b * g1b[:, None], W1c * g1c[:, None]], axis=0)
    b1 = jnp.concatenate([b1b * g1b + beta1b, b1c * g1c + beta1c])[:, None]
    pts_t = jnp.transpose(ctr_preds, (0, 2, 1))

    grid = (B, N // N_BLK)
    out_shapes = (
        jax.ShapeDtypeStruct((B, CODE_SIZE, N), jnp.float32),
        jax.ShapeDtypeStruct((B, 3, N), jnp.float32),
        jax.ShapeDtypeStruct((B, 7, N), jnp.float32),
    )
    full = lambda shape: pl.BlockSpec(shape, lambda b, n: (0,) * len(shape))
    ctr_box_preds, pt_cls_preds, pt_box_preds = pl.pallas_call(
        _fused_kernel,
        grid=grid,
        in_specs=[
            pl.BlockSpec(memory_space=pltpu.SMEM),
            pl.BlockSpec((1, C, N_BLK), lambda b, n: (b, 0, n)),
            pl.BlockSpec((1, 3, N_BLK), lambda b, n: (b, 0, n)),
            full((2 * MID, C)),
            full((2 * MID, 1)),
            full((CODE_SIZE, MID)),
            full((CODE_SIZE, 1)),
            full((3, MID)),
            full((3, 1)),
        ],
        out_specs=[
            pl.BlockSpec((1, CODE_SIZE, N_BLK), lambda b, n: (b, 0, n)),
            pl.BlockSpec((1, 3, N_BLK), lambda b, n: (b, 0, n)),
            pl.BlockSpec((1, 7, N_BLK), lambda b, n: (b, 0, n)),
        ],
        out_shape=out_shapes,
        compiler_params=pltpu.CompilerParams(
            dimension_semantics=("parallel", "parallel")),
    )(mean_size, ctr_feats, pts_t, w1, b1, W2b, b2b[:, None], W2c, b2c[:, None])
    return (jnp.transpose(ctr_box_preds, (0, 2, 1)),
            jnp.transpose(pt_cls_preds, (0, 2, 1)),
            jnp.transpose(pt_box_preds, (0, 2, 1)))
